# baseline (speedup 1.0000x reference)
import jax
import jax.numpy as jnp
from jax import lax
from jax.experimental import pallas as pl
from jax.experimental.pallas import tpu as pltpu
from jax.experimental.pallas import tpu_sc as plsc

WORD_DIM = 64
HIDDEN = 256
NTGT = 128
NKEY = 1000
BS, LS, MC = 4096, 50, 16
NPOS = BS * LS
NW = 32
PER_W = NPOS // NW
NVEC = PER_W // 16
CAPW = 128
NSLOT = NW * CAPW
CH = 128
NCH = PER_W // CH


def _wid():
    return lax.axis_index("s") * 2 + lax.axis_index("c")


def _sc_mask_compact_body(inp_hbm, maskf_hbm, tgtw_hbm, emb_hbm, lut_hbm,
                          gum_hbm,
                          idx_hbm, wordm_hbm, xm_hbm, gm_hbm,
                          lutm_hbm, luttgt_hbm, counts_hbm,
                          inp_v, maskf_v, idxl_v, wordm_v,
                          xm_v, gm_v, lutm_v, tg_v, cnt_v,
                          sem_e, sem_g, sem_l):
    wid = _wid()
    base = wid * PER_W
    pltpu.sync_copy(inp_hbm.at[pl.ds(base, PER_W)], inp_v)
    pltpu.sync_copy(maskf_hbm.at[pl.ds(base, PER_W)], maskf_v)

    iota16 = lax.iota(jnp.int32, 16)

    def blk(j, carry):
        cnt, punm = carry
        found = maskf_v[pl.ds(j * 16, 16)] > 0.5
        pos = base + j * 16 + iota16
        foundi = found.astype(jnp.int32)
        tgt = jnp.minimum(cnt + plsc.cumsum(foundi) - 1, CAPW - 1)
        plsc.store_scatter(idxl_v, [tgt], pos, mask=found)
        new_cnt = cnt + jnp.sum(foundi)
        unm_cand = jnp.min(jnp.where(found, NPOS, pos))
        new_punm = jnp.where(punm < 0,
                             jnp.where(unm_cand < NPOS, unm_cand, punm),
                             punm)
        return new_cnt, new_punm

    cnt, punm = lax.fori_loop(0, NVEC, blk, (jnp.int32(0), jnp.int32(-1)))
    punm = jnp.where(punm < 0, base, punm)

    for g in range(CAPW // 16):
        sl = g * 16 + iota16
        cur = idxl_v[pl.ds(g * 16, 16)]
        idx = jnp.where(sl < cnt, cur, jnp.full((16,), punm))
        idx = jnp.clip(idx, 0, NPOS - 1)
        idxl_v[pl.ds(g * 16, 16)] = idx
        wordm_v[pl.ds(g * 16, 16)] = plsc.load_gather(inp_v, [idx - base])

    pltpu.async_copy(emb_hbm.at[wordm_v], xm_v, sem_e)
    pltpu.async_copy(gum_hbm.at[idxl_v], gm_v, sem_g)
    cp_l = pltpu.async_copy(lut_hbm.at[wordm_v], lutm_v, sem_l)
    cp_l.wait()
    pltpu.make_async_copy(emb_hbm.at[wordm_v], xm_v, sem_e).wait()
    pltpu.make_async_copy(gum_hbm.at[idxl_v], gm_v, sem_g).wait()

    sbase = wid * CAPW
    pltpu.sync_copy(idxl_v, idx_hbm.at[pl.ds(sbase, CAPW)])
    pltpu.sync_copy(wordm_v, wordm_hbm.at[pl.ds(sbase, CAPW)])
    pltpu.sync_copy(xm_v, xm_hbm.at[pl.ds(sbase, CAPW)])
    pltpu.sync_copy(gm_v, gm_hbm.at[pl.ds(sbase, CAPW)])
    pltpu.sync_copy(lutm_v, lutm_hbm.at[pl.ds(sbase, CAPW)])
    cnt_v[...] = jnp.full((16,), cnt)
    pltpu.sync_copy(cnt_v, counts_hbm.at[wid])

    @pl.when(wid == 0)
    def _():
        pltpu.sync_copy(tgtw_hbm, tg_v)
        pltpu.async_copy(lut_hbm.at[tg_v], lutm_v, sem_l).wait()
        pltpu.sync_copy(lutm_v, luttgt_hbm)


def _sc_mask_compact(inp_flat, maskf_flat, tgtwords, emb_table, lut, gum):
    return pl.kernel(
        _sc_mask_compact_body,
        out_type=[
            jax.ShapeDtypeStruct((NSLOT,), jnp.int32),
            jax.ShapeDtypeStruct((NSLOT,), jnp.int32),
            jax.ShapeDtypeStruct((NSLOT, WORD_DIM), jnp.float32),
            jax.ShapeDtypeStruct((NSLOT, NTGT), jnp.float32),
            jax.ShapeDtypeStruct((NSLOT, MC), jnp.int32),
            jax.ShapeDtypeStruct((NTGT, MC), jnp.int32),
            jax.ShapeDtypeStruct((NW, 16), jnp.int32),
        ],
        mesh=plsc.VectorSubcoreMesh(core_axis_name="c", subcore_axis_name="s"),
        compiler_params=pltpu.CompilerParams(use_tc_tiling_on_sc=False, needs_layout_passes=False),
        scratch_types=[
            pltpu.VMEM((PER_W,), jnp.int32),
            pltpu.VMEM((PER_W,), jnp.float32),
            pltpu.VMEM((CAPW,), jnp.int32),
            pltpu.VMEM((CAPW,), jnp.int32),
            pltpu.VMEM((CAPW, WORD_DIM), jnp.float32),
            pltpu.VMEM((CAPW, NTGT), jnp.float32),
            pltpu.VMEM((CAPW, MC), jnp.int32),
            pltpu.VMEM((NTGT,), jnp.int32),
            pltpu.VMEM((16,), jnp.int32),
            pltpu.SemaphoreType.DMA,
            pltpu.SemaphoreType.DMA,
            pltpu.SemaphoreType.DMA,
        ],
    )(inp_flat, maskf_flat, tgtwords, emb_table, lut, gum)


GB = 64
GP = GB * LS
KWPAD = 1024
MROW = NPOS // 128
MBLK = GP // 128


def _tc_gumlin_body(g3_ref, glin_ref):
    glin_ref[...] = g3_ref[...].reshape(GP, NTGT)


def _tc_gumlin(gum3):
    return pl.pallas_call(
        _tc_gumlin_body,
        grid=(BS // GB,),
        in_specs=[pl.BlockSpec((GB, LS, NTGT), lambda b: (b, 0, 0))],
        out_specs=pl.BlockSpec((GP, NTGT), lambda b: (b, 0)),
        out_shape=jax.ShapeDtypeStruct((NPOS, NTGT), jnp.float32),
    )(gum3)


MB2 = 40


def _tc_mask_body(inp_ref, kw_ref, maskf_ref):
    words = inp_ref[...]
    kw = kw_ref[...].reshape(KWPAD)
    hit = words[:, :, None] == kw[None, None, :]
    maskf_ref[...] = jnp.any(hit, axis=2).astype(jnp.float32)


def _tc_mask(inp2, kw_p):
    return pl.pallas_call(
        _tc_mask_body,
        grid=(MROW // MB2,),
        in_specs=[
            pl.BlockSpec((MB2, 128), lambda b: (b, 0)),
            pl.BlockSpec((1, KWPAD), lambda b: (0, 0)),
        ],
        out_specs=pl.BlockSpec((MB2, 128), lambda b: (b, 0)),
        out_shape=jax.ShapeDtypeStruct((MROW, 128), jnp.float32),
    )(inp2, kw_p)


def _tc_slots_body(xm_ref, gm_ref, wordm_ref, lutm_ref, luttgt_ref, tgtw_ref,
                   tgt_ref, w1_ref, b1_ref, w2_ref, b2_ref, valid_ref,
                   word_ref, wemb_ref, char_ref, logp_ref, loss_ref):
    x = xm_ref[...]
    h1 = jnp.maximum(
        jnp.dot(x, w1_ref[...], preferred_element_type=jnp.float32)
        + b1_ref[...], 0.0)
    logits = (jnp.dot(h1, w2_ref[...], preferred_element_type=jnp.float32)
              + b2_ref[...])

    mx0 = jnp.max(logits, axis=-1, keepdims=True)
    e0 = jnp.exp(logits - mx0)
    p0 = e0 / jnp.sum(e0, axis=-1, keepdims=True)
    ent = -jnp.sum(p0 * jnp.log(p0 + 1e-12), axis=-1, keepdims=True)

    z = logits + gm_ref[...]
    mz = jnp.max(z, axis=-1, keepdims=True)
    ez = jnp.exp(z - mz)
    sft = ez / jnp.sum(ez, axis=-1, keepdims=True)

    mxs = jnp.max(sft, axis=-1, keepdims=True)
    iota = lax.broadcasted_iota(jnp.int32, (NSLOT, NTGT), 1)
    y = jnp.min(jnp.where(sft == mxs, iota, NTGT), axis=-1, keepdims=True)
    onehot = iota == y
    y_words = jnp.sum(jnp.where(onehot, tgtw_ref[...], 0),
                      axis=-1, keepdims=True)
    charm = jnp.dot(onehot.astype(jnp.float32), luttgt_ref[...],
                    preferred_element_type=jnp.float32)
    x_emb = jnp.dot(sft, tgt_ref[...],
                    preferred_element_type=jnp.float32)

    validf = valid_ref[...]
    valid = validf > 0.5
    word_ref[...] = jnp.where(valid, y_words, wordm_ref[...])
    wemb_ref[...] = jnp.where(valid, x_emb, x)
    char_ref[...] = jnp.where(valid, charm.astype(jnp.int32), lutm_ref[...])
    logp_ref[...] = jnp.where(valid, mx0, 0.0)
    nsp = jnp.maximum(jnp.sum(validf), 1.0)
    loss_ref[...] = (jnp.sum(ent * validf) / nsp * 0.03).reshape(1, 1)


def _tc_slots(xm, gm, wordm, lutm, luttgt_f, tgtw, tgt_table,
              W1, b1, W2, b2, validf):
    return pl.pallas_call(
        _tc_slots_body,
        out_shape=[
            jax.ShapeDtypeStruct((NSLOT, 1), jnp.int32),
            jax.ShapeDtypeStruct((NSLOT, WORD_DIM), jnp.float32),
            jax.ShapeDtypeStruct((NSLOT, MC), jnp.int32),
            jax.ShapeDtypeStruct((NSLOT, 1), jnp.float32),
            jax.ShapeDtypeStruct((1, 1), jnp.float32),
        ],
    )(xm, gm, wordm, lutm, luttgt_f, tgtw, tgt_table, W1, b1, W2, b2, validf)


def _sc_dense_scatter_body(inp_hbm, emb_hbm, lut_hbm, idx_hbm,
                           wordv_hbm, wembv_hbm, charv_hbm, logpv_hbm,
                           word_hbm, wemb_hbm, char_hbm, logp_hbm,
                           inp_v, idxb0_v, idxb1_v, rows0_v, rows1_v,
                           lrows0_v, lrows1_v, z_v,
                           fidx_v, fword_v, fwemb_v, fchar_v, flogp_v,
                           sem_e0, sem_l0, sem_e1, sem_l1):
    wid = _wid()
    base = wid * PER_W
    pltpu.sync_copy(inp_hbm.at[pl.ds(base, PER_W)], inp_v)

    def fill_zeros(j, carry):
        z_v[pl.ds(j * 16, 16)] = jnp.zeros((16,), jnp.float32)
        return carry

    lax.fori_loop(0, PER_W // 16, fill_zeros, 0)
    pltpu.sync_copy(z_v, logp_hbm.at[pl.ds(base, PER_W)])
    pltpu.sync_copy(inp_v, word_hbm.at[pl.ds(base, PER_W)])

    bufs = ((idxb0_v, rows0_v, lrows0_v, sem_e0, sem_l0),
            (idxb1_v, rows1_v, lrows1_v, sem_e1, sem_l1))

    def issue(c, buf):
        idxb, rows, lrows, sem_e, sem_l = buf
        off = c * CH
        for g in range(CH // 16):
            idxb[pl.ds(g * 16, 16)] = inp_v[pl.ds(off + g * 16, 16)]
        pltpu.async_copy(emb_hbm.at[idxb], rows, sem_e)
        pltpu.async_copy(lut_hbm.at[idxb], lrows, sem_l)

    def drain(c, buf):
        idxb, rows, lrows, sem_e, sem_l = buf
        pltpu.make_async_copy(emb_hbm.at[idxb], rows, sem_e).wait()
        pltpu.make_async_copy(lut_hbm.at[idxb], lrows, sem_l).wait()
        off = c * CH
        pltpu.sync_copy(rows, wemb_hbm.at[pl.ds(base + off, CH)])
        pltpu.sync_copy(lrows, char_hbm.at[pl.ds(base + off, CH)])

    issue(0, bufs[0])

    def step(i, carry):
        c0 = 2 * i
        issue(c0 + 1, bufs[1])
        drain(c0, bufs[0])

        @pl.when(i < NCH // 2 - 1)
        def _():
            issue(c0 + 2, bufs[0])

        drain(c0 + 1, bufs[1])
        return carry

    lax.fori_loop(0, NCH // 2, step, 0)

    sbase = wid * CAPW
    pltpu.sync_copy(idx_hbm.at[pl.ds(sbase, CAPW)], fidx_v)
    pltpu.sync_copy(wordv_hbm.at[pl.ds(sbase, CAPW)], fword_v)
    pltpu.sync_copy(wembv_hbm.at[pl.ds(sbase, CAPW)], fwemb_v)
    pltpu.sync_copy(charv_hbm.at[pl.ds(sbase, CAPW)], fchar_v)
    pltpu.sync_copy(logpv_hbm.at[pl.ds(sbase, CAPW)], flogp_v)

    pltpu.async_copy(fwemb_v, wemb_hbm.at[fidx_v], sem_e0)
    cp2 = pltpu.async_copy(fchar_v, char_hbm.at[fidx_v], sem_l0)
    cp2.wait()
    pltpu.make_async_copy(fwemb_v, wemb_hbm.at[fidx_v], sem_e0).wait()
    pltpu.async_copy(fword_v, word_hbm.at[fidx_v], sem_e1).wait()
    pltpu.async_copy(flogp_v, logp_hbm.at[fidx_v], sem_l1).wait()


def _sc_dense_scatter(inp_flat, emb_table, lut, idx, wordv, wembv, charv,
                      logpv):
    return pl.kernel(
        _sc_dense_scatter_body,
        out_type=[
            jax.ShapeDtypeStruct((NPOS,), jnp.int32),
            jax.ShapeDtypeStruct((NPOS, WORD_DIM), jnp.float32),
            jax.ShapeDtypeStruct((NPOS, MC), jnp.int32),
            jax.ShapeDtypeStruct((NPOS,), jnp.float32),
        ],
        mesh=plsc.VectorSubcoreMesh(core_axis_name="c", subcore_axis_name="s"),
        compiler_params=pltpu.CompilerParams(use_tc_tiling_on_sc=False, needs_layout_passes=False),
        scratch_types=[
            pltpu.VMEM((PER_W,), jnp.int32),
            pltpu.VMEM((CH,), jnp.int32),
            pltpu.VMEM((CH,), jnp.int32),
            pltpu.VMEM((CH, WORD_DIM), jnp.float32),
            pltpu.VMEM((CH, WORD_DIM), jnp.float32),
            pltpu.VMEM((CH, MC), jnp.int32),
            pltpu.VMEM((CH, MC), jnp.int32),
            pltpu.VMEM((PER_W,), jnp.float32),
            pltpu.VMEM((CAPW,), jnp.int32),
            pltpu.VMEM((CAPW,), jnp.int32),
            pltpu.VMEM((CAPW, WORD_DIM), jnp.float32),
            pltpu.VMEM((CAPW, MC), jnp.int32),
            pltpu.VMEM((CAPW,), jnp.float32),
            pltpu.SemaphoreType.DMA,
            pltpu.SemaphoreType.DMA,
            pltpu.SemaphoreType.DMA,
            pltpu.SemaphoreType.DMA,
        ],
    )(inp_flat, emb_table, lut, idx, wordv, wembv, charv, logpv)


def kernel(inp_word, inp_char, inp_pos, keywords, tgtwords, lut,
           emb_table, tgt_table, W1, b1, W2, b2, gumbel_noise):
    inp_flat = inp_word.reshape(NPOS)
    kw_p = jnp.full((KWPAD,), -1, jnp.int32).at[:NKEY].set(keywords)
    gum = _tc_gumlin(gumbel_noise)
    maskf2 = _tc_mask(inp_flat.reshape(MROW, 128), kw_p.reshape(1, KWPAD))
    maskf = maskf2.reshape(NPOS)

    (idx, wordm, xm, gm, lutm, luttgt, counts) = _sc_mask_compact(
        inp_flat, maskf, tgtwords, emb_table, lut, gum)

    validf = (lax.broadcasted_iota(jnp.int32, (NW, CAPW), 1)
              < counts[:, :1]).astype(jnp.float32).reshape(NSLOT, 1)

    wordv, wembv, charv, logpv, loss = _tc_slots(
        xm, gm, wordm.reshape(NSLOT, 1), lutm, luttgt.astype(jnp.float32),
        tgtwords.reshape(1, NTGT), tgt_table,
        W1, b1.reshape(1, HIDDEN), W2, b2.reshape(1, NTGT), validf)

    word, wemb, char, logp = _sc_dense_scatter(
        inp_flat, emb_table, lut, idx, wordv.reshape(NSLOT), wembv, charv,
        logpv.reshape(NSLOT))

    return (maskf.reshape(BS, LS), word.reshape(BS, LS),
            wemb.reshape(BS, LS, WORD_DIM), char.reshape(BS, LS, MC),
            logp.reshape(BS, LS), loss[0, 0])


# speedup vs baseline: 1.0125x; 1.0125x over previous
import jax
import jax.numpy as jnp
from jax import lax
from jax.experimental import pallas as pl
from jax.experimental.pallas import tpu as pltpu
from jax.experimental.pallas import tpu_sc as plsc

WORD_DIM = 64
HIDDEN = 256
NTGT = 128
NKEY = 1000
BS, LS, MC = 4096, 50, 16
NPOS = BS * LS
NW = 32
PER_W = NPOS // NW
NVEC = PER_W // 16
CAPW = 128
NSLOT = NW * CAPW
CH = 128
NCH = PER_W // CH


def _wid():
    return lax.axis_index("s") * 2 + lax.axis_index("c")


def _sc_mask_compact_body(inp_hbm, kw_hbm, tgtw_hbm, emb_hbm, lut_hbm, gum_hbm,
                          maskf_hbm, idx_hbm, wordm_hbm, xm_hbm, gm_hbm,
                          lutm_hbm, luttgt_hbm, counts_hbm,
                          inp_v, kw_v, maskf_v, idxl_v, wordm_v,
                          xm_v, gm_v, lutm_v, tg_v, cnt_v,
                          sem_e, sem_g, sem_l):
    wid = _wid()
    base = wid * PER_W
    pltpu.sync_copy(inp_hbm.at[pl.ds(base, PER_W)], inp_v)
    pltpu.sync_copy(kw_hbm, kw_v)

    iota16 = lax.iota(jnp.int32, 16)

    def blk(j, carry):
        cnt, punm = carry
        a = inp_v[pl.ds(j * 16, 16)]
        lo = jnp.zeros((16,), jnp.int32)
        hi = jnp.full((16,), NKEY, jnp.int32)
        for _ in range(10):
            mid = (lo + hi) >> 1
            kv = plsc.load_gather(kw_v, [mid])
            lt = kv < a
            lo = jnp.where(lt, mid + 1, lo)
            hi = jnp.where(lt, hi, mid)
        ins = jnp.minimum(lo, NKEY - 1)
        kv = plsc.load_gather(kw_v, [ins])
        found = kv == a
        maskf_v[pl.ds(j * 16, 16)] = jnp.where(found, 1.0, 0.0)
        pos = base + j * 16 + iota16
        foundi = found.astype(jnp.int32)
        tgt = jnp.minimum(cnt + plsc.cumsum(foundi) - 1, CAPW - 1)
        plsc.store_scatter(idxl_v, [tgt], pos, mask=found)
        new_cnt = cnt + jnp.sum(foundi)
        unm_cand = jnp.min(jnp.where(found, NPOS, pos))
        new_punm = jnp.where(punm < 0,
                             jnp.where(unm_cand < NPOS, unm_cand, punm),
                             punm)
        return new_cnt, new_punm

    cnt, punm = lax.fori_loop(0, NVEC, blk, (jnp.int32(0), jnp.int32(-1)))
    punm = jnp.where(punm < 0, base, punm)

    for g in range(CAPW // 16):
        sl = g * 16 + iota16
        cur = idxl_v[pl.ds(g * 16, 16)]
        idx = jnp.where(sl < cnt, cur, jnp.full((16,), punm))
        idx = jnp.clip(idx, 0, NPOS - 1)
        idxl_v[pl.ds(g * 16, 16)] = idx
        wordm_v[pl.ds(g * 16, 16)] = plsc.load_gather(inp_v, [idx - base])

    pltpu.async_copy(emb_hbm.at[wordm_v], xm_v, sem_e)
    pltpu.async_copy(gum_hbm.at[idxl_v], gm_v, sem_g)
    cp_l = pltpu.async_copy(lut_hbm.at[wordm_v], lutm_v, sem_l)
    cp_l.wait()
    pltpu.make_async_copy(emb_hbm.at[wordm_v], xm_v, sem_e).wait()
    pltpu.make_async_copy(gum_hbm.at[idxl_v], gm_v, sem_g).wait()

    sbase = wid * CAPW
    pltpu.sync_copy(maskf_v, maskf_hbm.at[pl.ds(base, PER_W)])
    pltpu.sync_copy(idxl_v, idx_hbm.at[pl.ds(sbase, CAPW)])
    pltpu.sync_copy(wordm_v, wordm_hbm.at[pl.ds(sbase, CAPW)])
    pltpu.sync_copy(xm_v, xm_hbm.at[pl.ds(sbase, CAPW)])
    pltpu.sync_copy(gm_v, gm_hbm.at[pl.ds(sbase, CAPW)])
    pltpu.sync_copy(lutm_v, lutm_hbm.at[pl.ds(sbase, CAPW)])
    cnt_v[...] = jnp.full((16,), cnt)
    pltpu.sync_copy(cnt_v, counts_hbm.at[wid])

    @pl.when(wid == 0)
    def _():
        pltpu.sync_copy(tgtw_hbm, tg_v)
        pltpu.async_copy(lut_hbm.at[tg_v], lutm_v, sem_l).wait()
        pltpu.sync_copy(lutm_v, luttgt_hbm)


def _sc_mask_compact(inp_flat, keywords, tgtwords, emb_table, lut, gum):
    return pl.kernel(
        _sc_mask_compact_body,
        out_type=[
            jax.ShapeDtypeStruct((NPOS,), jnp.float32),
            jax.ShapeDtypeStruct((NSLOT,), jnp.int32),
            jax.ShapeDtypeStruct((NSLOT,), jnp.int32),
            jax.ShapeDtypeStruct((NSLOT, WORD_DIM), jnp.float32),
            jax.ShapeDtypeStruct((NSLOT, NTGT), jnp.float32),
            jax.ShapeDtypeStruct((NSLOT, MC), jnp.int32),
            jax.ShapeDtypeStruct((NTGT, MC), jnp.int32),
            jax.ShapeDtypeStruct((NW, 16), jnp.int32),
        ],
        mesh=plsc.VectorSubcoreMesh(core_axis_name="c", subcore_axis_name="s"),
        compiler_params=pltpu.CompilerParams(use_tc_tiling_on_sc=False, needs_layout_passes=False),
        scratch_types=[
            pltpu.VMEM((PER_W,), jnp.int32),
            pltpu.VMEM((NKEY,), jnp.int32),
            pltpu.VMEM((PER_W,), jnp.float32),
            pltpu.VMEM((CAPW,), jnp.int32),
            pltpu.VMEM((CAPW,), jnp.int32),
            pltpu.VMEM((CAPW, WORD_DIM), jnp.float32),
            pltpu.VMEM((CAPW, NTGT), jnp.float32),
            pltpu.VMEM((CAPW, MC), jnp.int32),
            pltpu.VMEM((NTGT,), jnp.int32),
            pltpu.VMEM((16,), jnp.int32),
            pltpu.SemaphoreType.DMA,
            pltpu.SemaphoreType.DMA,
            pltpu.SemaphoreType.DMA,
        ],
    )(inp_flat, keywords, tgtwords, emb_table, lut, gum)


GB = 64
GP = GB * LS


def _tc_gumlin_body(g3_ref, out_ref):
    out_ref[...] = g3_ref[...].reshape(GP, NTGT)


def _tc_gumlin(gum3):
    return pl.pallas_call(
        _tc_gumlin_body,
        grid=(BS // GB,),
        in_specs=[pl.BlockSpec((GB, LS, NTGT), lambda b: (b, 0, 0))],
        out_specs=pl.BlockSpec((GP, NTGT), lambda b: (b, 0)),
        out_shape=jax.ShapeDtypeStruct((NPOS, NTGT), jnp.float32),
    )(gum3)


def _tc_slots_body(xm_ref, gm_ref, wordm_ref, lutm_ref, luttgt_ref, tgtw_ref,
                   tgt_ref, w1_ref, b1_ref, w2_ref, b2_ref, valid_ref,
                   word_ref, wemb_ref, char_ref, logp_ref, loss_ref):
    x = xm_ref[...]
    h1 = jnp.maximum(
        jnp.dot(x, w1_ref[...], preferred_element_type=jnp.float32)
        + b1_ref[...], 0.0)
    logits = (jnp.dot(h1, w2_ref[...], preferred_element_type=jnp.float32)
              + b2_ref[...])

    mx0 = jnp.max(logits, axis=-1, keepdims=True)
    e0 = jnp.exp(logits - mx0)
    p0 = e0 / jnp.sum(e0, axis=-1, keepdims=True)
    ent = -jnp.sum(p0 * jnp.log(p0 + 1e-12), axis=-1, keepdims=True)

    z = logits + gm_ref[...]
    mz = jnp.max(z, axis=-1, keepdims=True)
    ez = jnp.exp(z - mz)
    sft = ez / jnp.sum(ez, axis=-1, keepdims=True)

    mxs = jnp.max(sft, axis=-1, keepdims=True)
    iota = lax.broadcasted_iota(jnp.int32, (NSLOT, NTGT), 1)
    y = jnp.min(jnp.where(sft == mxs, iota, NTGT), axis=-1, keepdims=True)
    onehot = iota == y
    y_words = jnp.sum(jnp.where(onehot, tgtw_ref[...], 0),
                      axis=-1, keepdims=True)
    charm = jnp.dot(onehot.astype(jnp.float32), luttgt_ref[...],
                    preferred_element_type=jnp.float32)
    x_emb = jnp.dot(sft, tgt_ref[...],
                    preferred_element_type=jnp.float32)

    validf = valid_ref[...]
    valid = validf > 0.5
    word_ref[...] = jnp.where(valid, y_words, wordm_ref[...])
    wemb_ref[...] = jnp.where(valid, x_emb, x)
    char_ref[...] = jnp.where(valid, charm.astype(jnp.int32), lutm_ref[...])
    logp_ref[...] = jnp.where(valid, mx0, 0.0)
    nsp = jnp.maximum(jnp.sum(validf), 1.0)
    loss_ref[...] = (jnp.sum(ent * validf) / nsp * 0.03).reshape(1, 1)


def _tc_slots(xm, gm, wordm, lutm, luttgt_f, tgtw, tgt_table,
              W1, b1, W2, b2, validf):
    return pl.pallas_call(
        _tc_slots_body,
        out_shape=[
            jax.ShapeDtypeStruct((NSLOT, 1), jnp.int32),
            jax.ShapeDtypeStruct((NSLOT, WORD_DIM), jnp.float32),
            jax.ShapeDtypeStruct((NSLOT, MC), jnp.int32),
            jax.ShapeDtypeStruct((NSLOT, 1), jnp.float32),
            jax.ShapeDtypeStruct((1, 1), jnp.float32),
        ],
    )(xm, gm, wordm, lutm, luttgt_f, tgtw, tgt_table, W1, b1, W2, b2, validf)


def _sc_dense_scatter_body(inp_hbm, emb_hbm, lut_hbm, idx_hbm,
                           wordv_hbm, wembv_hbm, charv_hbm, logpv_hbm,
                           word_hbm, wemb_hbm, char_hbm, logp_hbm,
                           inp_v, idxb0_v, idxb1_v, rows0_v, rows1_v,
                           lrows0_v, lrows1_v, z_v,
                           fidx_v, fword_v, fwemb_v, fchar_v, flogp_v,
                           sem_e0, sem_l0, sem_e1, sem_l1):
    wid = _wid()
    base = wid * PER_W
    pltpu.sync_copy(inp_hbm.at[pl.ds(base, PER_W)], inp_v)

    def fill_zeros(j, carry):
        z_v[pl.ds(j * 16, 16)] = jnp.zeros((16,), jnp.float32)
        return carry

    lax.fori_loop(0, PER_W // 16, fill_zeros, 0)
    pltpu.sync_copy(z_v, logp_hbm.at[pl.ds(base, PER_W)])
    pltpu.sync_copy(inp_v, word_hbm.at[pl.ds(base, PER_W)])

    bufs = ((idxb0_v, rows0_v, lrows0_v, sem_e0, sem_l0),
            (idxb1_v, rows1_v, lrows1_v, sem_e1, sem_l1))

    def issue(c, buf):
        idxb, rows, lrows, sem_e, sem_l = buf
        off = c * CH
        for g in range(CH // 16):
            idxb[pl.ds(g * 16, 16)] = inp_v[pl.ds(off + g * 16, 16)]
        pltpu.async_copy(emb_hbm.at[idxb], rows, sem_e)
        pltpu.async_copy(lut_hbm.at[idxb], lrows, sem_l)

    def drain(c, buf):
        idxb, rows, lrows, sem_e, sem_l = buf
        pltpu.make_async_copy(emb_hbm.at[idxb], rows, sem_e).wait()
        pltpu.make_async_copy(lut_hbm.at[idxb], lrows, sem_l).wait()
        off = c * CH
        pltpu.sync_copy(rows, wemb_hbm.at[pl.ds(base + off, CH)])
        pltpu.sync_copy(lrows, char_hbm.at[pl.ds(base + off, CH)])

    issue(0, bufs[0])

    def step(i, carry):
        c0 = 2 * i
        issue(c0 + 1, bufs[1])
        drain(c0, bufs[0])

        @pl.when(i < NCH // 2 - 1)
        def _():
            issue(c0 + 2, bufs[0])

        drain(c0 + 1, bufs[1])
        return carry

    lax.fori_loop(0, NCH // 2, step, 0)

    sbase = wid * CAPW
    pltpu.sync_copy(idx_hbm.at[pl.ds(sbase, CAPW)], fidx_v)
    pltpu.sync_copy(wordv_hbm.at[pl.ds(sbase, CAPW)], fword_v)
    pltpu.sync_copy(wembv_hbm.at[pl.ds(sbase, CAPW)], fwemb_v)
    pltpu.sync_copy(charv_hbm.at[pl.ds(sbase, CAPW)], fchar_v)
    pltpu.sync_copy(logpv_hbm.at[pl.ds(sbase, CAPW)], flogp_v)

    pltpu.async_copy(fwemb_v, wemb_hbm.at[fidx_v], sem_e0)
    cp2 = pltpu.async_copy(fchar_v, char_hbm.at[fidx_v], sem_l0)
    cp2.wait()
    pltpu.make_async_copy(fwemb_v, wemb_hbm.at[fidx_v], sem_e0).wait()
    pltpu.async_copy(fword_v, word_hbm.at[fidx_v], sem_e1).wait()
    pltpu.async_copy(flogp_v, logp_hbm.at[fidx_v], sem_l1).wait()


def _sc_dense_scatter(inp_flat, emb_table, lut, idx, wordv, wembv, charv,
                      logpv):
    return pl.kernel(
        _sc_dense_scatter_body,
        out_type=[
            jax.ShapeDtypeStruct((NPOS,), jnp.int32),
            jax.ShapeDtypeStruct((NPOS, WORD_DIM), jnp.float32),
            jax.ShapeDtypeStruct((NPOS, MC), jnp.int32),
            jax.ShapeDtypeStruct((NPOS,), jnp.float32),
        ],
        mesh=plsc.VectorSubcoreMesh(core_axis_name="c", subcore_axis_name="s"),
        compiler_params=pltpu.CompilerParams(use_tc_tiling_on_sc=False, needs_layout_passes=False),
        scratch_types=[
            pltpu.VMEM((PER_W,), jnp.int32),
            pltpu.VMEM((CH,), jnp.int32),
            pltpu.VMEM((CH,), jnp.int32),
            pltpu.VMEM((CH, WORD_DIM), jnp.float32),
            pltpu.VMEM((CH, WORD_DIM), jnp.float32),
            pltpu.VMEM((CH, MC), jnp.int32),
            pltpu.VMEM((CH, MC), jnp.int32),
            pltpu.VMEM((PER_W,), jnp.float32),
            pltpu.VMEM((CAPW,), jnp.int32),
            pltpu.VMEM((CAPW,), jnp.int32),
            pltpu.VMEM((CAPW, WORD_DIM), jnp.float32),
            pltpu.VMEM((CAPW, MC), jnp.int32),
            pltpu.VMEM((CAPW,), jnp.float32),
            pltpu.SemaphoreType.DMA,
            pltpu.SemaphoreType.DMA,
            pltpu.SemaphoreType.DMA,
            pltpu.SemaphoreType.DMA,
        ],
    )(inp_flat, emb_table, lut, idx, wordv, wembv, charv, logpv)


def kernel(inp_word, inp_char, inp_pos, keywords, tgtwords, lut,
           emb_table, tgt_table, W1, b1, W2, b2, gumbel_noise):
    inp_flat = inp_word.reshape(NPOS)
    gum = _tc_gumlin(gumbel_noise)

    (maskf, idx, wordm, xm, gm, lutm, luttgt, counts) = _sc_mask_compact(
        inp_flat, keywords, tgtwords, emb_table, lut, gum)

    validf = (lax.broadcasted_iota(jnp.int32, (NW, CAPW), 1)
              < counts[:, :1]).astype(jnp.float32).reshape(NSLOT, 1)

    wordv, wembv, charv, logpv, loss = _tc_slots(
        xm, gm, wordm.reshape(NSLOT, 1), lutm, luttgt.astype(jnp.float32),
        tgtwords.reshape(1, NTGT), tgt_table,
        W1, b1.reshape(1, HIDDEN), W2, b2.reshape(1, NTGT), validf)

    word, wemb, char, logp = _sc_dense_scatter(
        inp_flat, emb_table, lut, idx, wordv.reshape(NSLOT), wembv, charv,
        logpv.reshape(NSLOT))

    return (maskf.reshape(BS, LS), word.reshape(BS, LS),
            wemb.reshape(BS, LS, WORD_DIM), char.reshape(BS, LS, MC),
            logp.reshape(BS, LS), loss[0, 0])
